# speedup vs baseline: 1.0270x; 1.0270x over previous
import functools
import jax
import jax.numpy as jnp
from jax.experimental import pallas as pl

N_NODES = 10000
D = 128
ROW_BLK = 1000


def _dense1_body(x_ref, agg_ref, w_ref, b_ref, lnw_ref, lnb_ref, eps_ref,
                 a_ref, o_ref):
    h = (1.0 + eps_ref[0, 0]) * x_ref[...] + agg_ref[...]
    h = jnp.dot(h, w_ref[...], preferred_element_type=jnp.float32) + b_ref[...]
    mu = jnp.mean(h, axis=-1, keepdims=True)
    var = jnp.mean((h - mu) ** 2, axis=-1, keepdims=True)
    h = (h - mu) * jax.lax.rsqrt(var + 1e-5) * lnw_ref[...] + lnb_ref[...]
    o_ref[...] = jnp.where(h > 0, h, a_ref[0, 0] * h)


def _dense2_body(h_ref, agg_ref, w_ref, b_ref, eps_ref, o_ref):
    t = (1.0 + eps_ref[0, 0]) * h_ref[...] + agg_ref[...]
    o_ref[...] = jnp.dot(t, w_ref[...], preferred_element_type=jnp.float32) \
        + b_ref[...]


def _dense1(x, agg, W1T, b1, ln_w, ln_b, eps1, prelu_a):
    grid = (N_NODES // ROW_BLK,)
    blk = pl.BlockSpec((ROW_BLK, D), lambda i: (i, 0))
    full = pl.BlockSpec((D, D), lambda i: (0, 0))
    vec = pl.BlockSpec((1, D), lambda i: (0, 0))
    sca = pl.BlockSpec((1, 1), lambda i: (0, 0))
    return pl.pallas_call(
        _dense1_body,
        grid=grid,
        in_specs=[blk, blk, full, vec, vec, vec, sca, sca],
        out_specs=blk,
        out_shape=jax.ShapeDtypeStruct((N_NODES, D), jnp.float32),
    )(x, agg, W1T, b1.reshape(1, D), ln_w.reshape(1, D), ln_b.reshape(1, D),
      eps1.reshape(1, 1), prelu_a.reshape(1, 1))


def _dense2(h, agg, W2T, b2, eps2):
    grid = (N_NODES // ROW_BLK,)
    blk = pl.BlockSpec((ROW_BLK, D), lambda i: (i, 0))
    full = pl.BlockSpec((D, D), lambda i: (0, 0))
    vec = pl.BlockSpec((1, D), lambda i: (0, 0))
    sca = pl.BlockSpec((1, 1), lambda i: (0, 0))
    return pl.pallas_call(
        _dense2_body,
        grid=grid,
        in_specs=[blk, blk, full, vec, sca],
        out_specs=blk,
        out_shape=jax.ShapeDtypeStruct((N_NODES, D), jnp.float32),
    )(h, agg, W2T, b2.reshape(1, D), eps2.reshape(1, 1))


def _seg_max(data, seg, n):
    m = jax.ops.segment_max(data, seg, num_segments=n)
    cnt = jax.ops.segment_sum(
        jnp.ones((seg.shape[0], 1), dtype=data.dtype), seg, num_segments=n)
    return jnp.where(cnt > 0, m, jnp.zeros_like(m))


@jax.jit
def kernel(x, edge_index, W1, b1, eps1, ln_w, ln_b, prelu_a, W2, b2, eps2):
    src = edge_index[0]
    dst = edge_index[1]
    agg1 = _seg_max(x[src], dst, N_NODES)
    h = _dense1(x, agg1, W1.T, b1, ln_w, ln_b, eps1, prelu_a)
    agg2 = _seg_max(h[src], dst, N_NODES)
    return _dense2(h, agg2, W2.T, b2, eps2)
